# baseline (speedup 1.0000x reference)
import functools

import jax
import jax.numpy as jnp
from jax import lax
from jax.experimental import pallas as pl
from jax.experimental.pallas import tpu as pltpu
from jax.experimental.pallas import tpu_sc as plsc

H = 224
W = 224
HW = H * W
C = 96
GAMMA = 0.1
NPAIR = 4
ROWS = HW // 128
PAD = 256
RCH = 3136
NRB = HW // RCH
NIMG = 6

SC_NC = 2
SC_NS = 16
SC_NW = SC_NC * SC_NS
NROWTOT = NPAIR * HW
BPW = NROWTOT // SC_NW
GCH = 128
NG = BPW // GCH


def _bf(v):
    return v.astype(jnp.bfloat16).astype(jnp.float32)


def _proj_body(pts_ref, coef_ref, idx_ref, msk_ref, cnt_ref):
    p = pl.program_id(0)
    x = _bf(pts_ref[0, 0])
    y = _bf(pts_ref[0, 1])
    z = _bf(pts_ref[0, 2])

    def crow(r):
        return (coef_ref[p, r, 0], coef_ref[p, r, 1], coef_ref[p, r, 2],
                coef_ref[p, r, 3])

    a0, a1, a2, a3 = crow(0)
    camx = a0 * x + a1 * y + a2 * z + a3
    a0, a1, a2, a3 = crow(1)
    camy = a0 * x + a1 * y + a2 * z + a3
    a0, a1, a2, a3 = crow(2)
    camz = a0 * x + a1 * y + a2 * z + a3
    denom = camz + 1e-8
    px_ = _bf(camx / denom)
    py_ = _bf(camy / denom)
    pz_ = _bf(camz / denom)
    i00, i01, i02, _ = crow(3)
    ndcx = i00 * px_ + i01 * py_ + i02 * pz_
    i10, i11, i12, _ = crow(4)
    ndcy = i10 * px_ + i11 * py_ + i12 * pz_
    valid = ((ndcx >= 0.0) & (ndcx < 1.0) & (ndcy >= 0.0) & (ndcy < 1.0)
             & (camz >= 0.0))
    pxi = jnp.clip(jnp.floor(ndcx * W).astype(jnp.int32), 0, W - 1)
    pyi = jnp.clip(jnp.floor(ndcy * H).astype(jnp.int32), 0, H - 1)
    base = coef_ref[p, 5, 0].astype(jnp.int32)
    idx_ref[...] = (base + pyi * W + pxi)[None]
    mf = valid.astype(jnp.float32)
    msk_ref[...] = mf[None]
    cnt_ref[...] = jnp.broadcast_to(jnp.sum(mf), (1, 1, 128))


def _project(pts, coef):
    return pl.pallas_call(
        _proj_body,
        grid=(NPAIR,),
        in_specs=[
            pl.BlockSpec((1, 3, ROWS, 128), lambda p: (p, 0, 0, 0)),
            pl.BlockSpec(memory_space=pltpu.SMEM),
        ],
        out_specs=[
            pl.BlockSpec((1, ROWS, 128), lambda p: (p, 0, 0)),
            pl.BlockSpec((1, ROWS, 128), lambda p: (p, 0, 0)),
            pl.BlockSpec((1, 1, 128), lambda p: (p, 0, 0)),
        ],
        out_shape=[
            jax.ShapeDtypeStruct((NPAIR, ROWS, 128), jnp.int32),
            jax.ShapeDtypeStruct((NPAIR, ROWS, 128), jnp.float32),
            jax.ShapeDtypeStruct((NPAIR, 1, 128), jnp.float32),
        ],
    )(pts, coef)


CP = 128


def _sc_gather_body(tbl_hbm, idx_hbm, out_hbm, idx_v, rows_a, rows_b,
                    sem_a, sem_b):
    wid = lax.axis_index("s") * SC_NC + lax.axis_index("c")
    ibase = wid * NG
    pltpu.sync_copy(idx_hbm.at[wid], idx_v)
    pltpu.make_async_copy(tbl_hbm.at[idx_v.at[0]], rows_a, sem_a).start()

    @pl.loop(0, (NG - 1) // 2)
    def _(i):
        g = 1 + 2 * i
        pltpu.make_async_copy(tbl_hbm.at[idx_v.at[g]], rows_b, sem_b).start()
        pltpu.make_async_copy(tbl_hbm.at[idx_v.at[g - 1]], rows_a, sem_a).wait()
        pltpu.sync_copy(rows_a, out_hbm.at[pl.ds((ibase + g - 1) * GCH, GCH)])
        pltpu.make_async_copy(tbl_hbm.at[idx_v.at[g + 1]], rows_a, sem_a).start()
        pltpu.make_async_copy(tbl_hbm.at[idx_v.at[g]], rows_b, sem_b).wait()
        pltpu.sync_copy(rows_b, out_hbm.at[pl.ds((ibase + g) * GCH, GCH)])

    pltpu.make_async_copy(tbl_hbm.at[idx_v.at[NG - 1]], rows_a, sem_a).wait()
    pltpu.sync_copy(rows_a, out_hbm.at[pl.ds((ibase + NG - 1) * GCH, GCH)])


def _sc_gather(table, gidx2d):
    k = pl.kernel(
        _sc_gather_body,
        out_type=jax.ShapeDtypeStruct((NROWTOT, CP), jnp.float32),
        mesh=plsc.VectorSubcoreMesh(core_axis_name="c", subcore_axis_name="s",
                                    num_cores=SC_NC, num_subcores=SC_NS),
        scratch_types=[
            pltpu.VMEM((NG, GCH), jnp.int32),
            pltpu.VMEM((GCH, CP), jnp.float32),
            pltpu.VMEM((GCH, CP), jnp.float32),
            pltpu.SemaphoreType.DMA,
            pltpu.SemaphoreType.DMA,
        ],
        compiler_params=pltpu.CompilerParams(use_tc_tiling_on_sc=True),
    )
    return k(table, gidx2d)


def _pad_body(src_ref, out_ref):
    out_ref[0, :, :C] = src_ref[0]
    out_ref[0, :, C:] = jnp.zeros((RCH, CP - C), jnp.float32)


def _pad128(src6):
    return pl.pallas_call(
        _pad_body,
        grid=(NIMG, NRB),
        in_specs=[pl.BlockSpec((1, RCH, C), lambda n, rb: (n, rb, 0))],
        out_specs=pl.BlockSpec((1, RCH, CP), lambda n, rb: (n, rb, 0)),
        out_shape=jax.ShapeDtypeStruct((NIMG, HW, CP), jnp.float32),
    )(src6)


def _combine_body(src_ref, gat_ref, msk_ref, scal_ref, out_ref):
    n = pl.program_id(0)
    s = scal_ref[n, 0]
    r = scal_ref[n, 1]
    x = src_ref[0].astype(jnp.float32)
    g = gat_ref[0, :, :C].astype(jnp.float32)
    m = msk_ref[0].astype(jnp.float32)
    out_ref[0] = ((x + (s * m) * g) * r).astype(jnp.bfloat16)


def _combine(src6, gath, mask, scal):
    def gmap(n, rb):
        i = n // 3
        j = n % 3
        p = i * 2 + jnp.where(j < 2, j, 0)
        return (p, rb, 0)

    return pl.pallas_call(
        _combine_body,
        grid=(NIMG, NRB),
        in_specs=[
            pl.BlockSpec((1, RCH, C), lambda n, rb: (n, rb, 0)),
            pl.BlockSpec((1, RCH, CP), gmap),
            pl.BlockSpec((1, RCH, 1), gmap),
            pl.BlockSpec(memory_space=pltpu.SMEM),
        ],
        out_specs=pl.BlockSpec((1, RCH, C), lambda n, rb: (n, rb, 0)),
        out_shape=jax.ShapeDtypeStruct((NIMG, HW, C), jnp.bfloat16),
    )(src6, gath, mask, scal)


def _erf(x):
    a1, a2, a3, a4, a5 = (0.254829592, -0.284496736, 1.421413741,
                          -1.453152027, 1.061405429)
    sgn = jnp.sign(x)
    ax = jnp.abs(x)
    t = 1.0 / (1.0 + 0.3275911 * ax)
    poly = t * (a1 + t * (a2 + t * (a3 + t * (a4 + t * a5))))
    return sgn * (1.0 - poly * jnp.exp(-ax * ax))


_TAPS = tuple((dy, dx) for dy in (-1, 0, 1) for dx in (-1, 0, 1))


HWP = HW + 2 * PAD


def _build_shifted(x_hbm, comb_ref, sem, n):
    zpad = jnp.zeros((PAD, C), jnp.bfloat16)
    for s in range(3):
        comb_ref[s, pl.ds(0, PAD), :] = zpad
        comb_ref[s, pl.ds(PAD + HW, PAD), :] = zpad
    cp = pltpu.make_async_copy(
        x_hbm.at[n], comb_ref.at[1, pl.ds(PAD, HW)], sem)
    cp.start()
    cp.wait()
    wc = lax.broadcasted_iota(jnp.int32, (RCH, 1), 0) % W
    for t in range(NRB):
        q0 = PAD + t * RCH
        ext = comb_ref[1, pl.ds(q0 - 16, RCH + 32), :]
        sm = lax.slice_in_dim(ext, 15, 15 + RCH, axis=0)
        comb_ref[0, pl.ds(q0, RCH), :] = jnp.where(
            wc == 0, jnp.bfloat16(0), sm)
        sp = lax.slice_in_dim(ext, 17, 17 + RCH, axis=0)
        comb_ref[2, pl.ds(q0, RCH), :] = jnp.where(
            wc == W - 1, jnp.bfloat16(0), sp)


def _conv_accum(comb_ref, w_ref, bias_ref, rb):
    base = PAD + rb * RCH
    acc = jnp.broadcast_to(bias_ref[...].astype(jnp.float32), (RCH, C))
    for dy in (-1, 0, 1):
        for dx in (-1, 0, 1):
            t = (dy + 1) * 3 + (dx + 1)
            xs = comb_ref[dx + 1, pl.ds(base + dy * W, RCH), :]
            acc = acc + jax.lax.dot_general(
                xs, w_ref[t], (((1,), (0,)), ((), ())),
                preferred_element_type=jnp.float32)
    return acc


def _conv1_body(x_ref, w_ref, bias_ref, out_ref, comb_ref, sem):
    n = pl.program_id(0)
    rb = pl.program_id(1)

    @pl.when(rb == 0)
    def _():
        _build_shifted(x_ref, comb_ref, sem, n)

    acc = _conv_accum(comb_ref, w_ref, bias_ref, rb)
    gelu = acc * 0.5 * (1.0 + _erf(acc * 0.7071067811865476))
    out_ref[0] = gelu.astype(jnp.bfloat16)


def _conv2_body(x_ref, w_ref, bias_ref, out_ref, comb_ref, sem):
    n = pl.program_id(0)
    rb = pl.program_id(1)

    @pl.when(rb == 0)
    def _():
        _build_shifted(x_ref, comb_ref, sem, n)

    acc = _conv_accum(comb_ref, w_ref, bias_ref, rb)
    out_ref[0] = acc


def _conv(x, wt, bias, body, out_dtype):
    return pl.pallas_call(
        body,
        grid=(NIMG, NRB),
        in_specs=[
            pl.BlockSpec(memory_space=pltpu.HBM),
            pl.BlockSpec((9, C, C), lambda n, rb: (0, 0, 0)),
            pl.BlockSpec((1, C), lambda n, rb: (0, 0)),
        ],
        out_specs=pl.BlockSpec((1, RCH, C), lambda n, rb: (n, rb, 0)),
        out_shape=jax.ShapeDtypeStruct((NIMG, HW, C), out_dtype),
        scratch_shapes=[pltpu.VMEM((3, HWP, C), jnp.bfloat16),
                        pltpu.SemaphoreType.DMA],
    )(x, wt, bias)


def kernel(means, depths, gs_feats, intrinsics, extrinsics,
           conv1_w, conv1_b, conv2_w, conv2_b):
    del depths
    b, v, h, w, c = gs_feats.shape

    w2c = jnp.linalg.inv(extrinsics)
    pi = jnp.array([0, 0, 1, 1])
    pk = jnp.array([1, 2, 1, 2])
    w2c_p = w2c[pi, pk]
    intr_p = intrinsics[pi, pk]
    w2c_b = w2c_p.astype(jnp.bfloat16).astype(jnp.float32)
    intr_b = intr_p.astype(jnp.bfloat16).astype(jnp.float32)
    coef = jnp.zeros((NPAIR, 6, 4), jnp.float32)
    coef = coef.at[:, 0:3, :].set(w2c_b[:, 0:3, :])
    coef = coef.at[:, 3, 0:3].set(intr_b[:, 0, :])
    coef = coef.at[:, 4, 0:3].set(intr_b[:, 1, :])
    tgt_base = ((pi * v + pk) * HW).astype(jnp.float32)
    coef = coef.at[:, 5, 0].set(tgt_base)

    pts = means[:, :2].reshape(NPAIR, HW, 3).transpose(0, 2, 1)
    pts = pts.reshape(NPAIR, 3, ROWS, 128)

    gidx, maskf, cnt = _project(pts, coef)

    table = _pad128(gs_feats.reshape(NIMG, HW, C)).reshape(b * v * HW, CP)
    gidx2d = gidx.reshape(SC_NW, NG, GCH)
    gath = _sc_gather(table, gidx2d)

    frac = cnt[:, 0, 0] / float(HW)
    s_pair = GAMMA * frac
    s_img = jnp.stack([s_pair[0], s_pair[1], jnp.zeros(()),
                       s_pair[2], s_pair[3], jnp.zeros(())])
    r_img = 1.0 / (1.0 + s_img)
    scal = jnp.stack([s_img, r_img], axis=1)

    src6 = gs_feats.reshape(NIMG, HW, C)
    gath3 = gath.reshape(NPAIR, HW, CP)
    mask3 = maskf.reshape(NPAIR, HW, 1)
    comb = _combine(src6, gath3, mask3, scal)

    w1 = conv1_w.transpose(2, 3, 1, 0).reshape(9, C, C).astype(jnp.bfloat16)
    w2 = conv2_w.transpose(2, 3, 1, 0).reshape(9, C, C).astype(jnp.bfloat16)
    b1 = conv1_b.reshape(1, C)
    b2 = conv2_b.reshape(1, C)
    mid = _conv(comb, w1, b1, _conv1_body, jnp.bfloat16)
    out = _conv(mid, w2, b2, _conv2_body, jnp.float32)

    return out.reshape(b, v, h, w, c)


# speedup vs baseline: 1.3841x; 1.3841x over previous
import functools

import jax
import jax.numpy as jnp
from jax import lax
from jax.experimental import pallas as pl
from jax.experimental.pallas import tpu as pltpu
from jax.experimental.pallas import tpu_sc as plsc

H = 224
W = 224
HW = H * W
C = 96
GAMMA = 0.1
NPAIR = 4
ROWS = HW // 128
PAD = 256
RCH = 3136
NRB = HW // RCH
NIMG = 6

SC_NC = 2
SC_NS = 16
SC_NW = SC_NC * SC_NS
NROWTOT = NPAIR * HW
BPW = NROWTOT // SC_NW
GCH = 128
NG = BPW // GCH


def _bf(v):
    return v.astype(jnp.bfloat16).astype(jnp.float32)


def _proj_body(pts_ref, coef_ref, idx_ref, msk_ref, cnt_ref):
    p = pl.program_id(0)
    x = _bf(pts_ref[0, 0])
    y = _bf(pts_ref[0, 1])
    z = _bf(pts_ref[0, 2])

    def crow(r):
        return (coef_ref[p, r, 0], coef_ref[p, r, 1], coef_ref[p, r, 2],
                coef_ref[p, r, 3])

    a0, a1, a2, a3 = crow(0)
    camx = a0 * x + a1 * y + a2 * z + a3
    a0, a1, a2, a3 = crow(1)
    camy = a0 * x + a1 * y + a2 * z + a3
    a0, a1, a2, a3 = crow(2)
    camz = a0 * x + a1 * y + a2 * z + a3
    denom = camz + 1e-8
    px_ = _bf(camx / denom)
    py_ = _bf(camy / denom)
    pz_ = _bf(camz / denom)
    i00, i01, i02, _ = crow(3)
    ndcx = i00 * px_ + i01 * py_ + i02 * pz_
    i10, i11, i12, _ = crow(4)
    ndcy = i10 * px_ + i11 * py_ + i12 * pz_
    valid = ((ndcx >= 0.0) & (ndcx < 1.0) & (ndcy >= 0.0) & (ndcy < 1.0)
             & (camz >= 0.0))
    pxi = jnp.clip(jnp.floor(ndcx * W).astype(jnp.int32), 0, W - 1)
    pyi = jnp.clip(jnp.floor(ndcy * H).astype(jnp.int32), 0, H - 1)
    base = coef_ref[p, 5, 0].astype(jnp.int32)
    idx_ref[...] = (base + pyi * W + pxi)[None]
    mf = valid.astype(jnp.float32)
    msk_ref[...] = mf[None]
    cnt_ref[...] = jnp.broadcast_to(jnp.sum(mf), (1, 1, 128))


def _project(pts, coef):
    return pl.pallas_call(
        _proj_body,
        grid=(NPAIR,),
        in_specs=[
            pl.BlockSpec((1, 3, ROWS, 128), lambda p: (p, 0, 0, 0)),
            pl.BlockSpec(memory_space=pltpu.SMEM),
        ],
        out_specs=[
            pl.BlockSpec((1, ROWS, 128), lambda p: (p, 0, 0)),
            pl.BlockSpec((1, ROWS, 128), lambda p: (p, 0, 0)),
            pl.BlockSpec((1, 1, 128), lambda p: (p, 0, 0)),
        ],
        out_shape=[
            jax.ShapeDtypeStruct((NPAIR, ROWS, 128), jnp.int32),
            jax.ShapeDtypeStruct((NPAIR, ROWS, 128), jnp.float32),
            jax.ShapeDtypeStruct((NPAIR, 1, 128), jnp.float32),
        ],
    )(pts, coef)


CP = 128


def _sc_gather_body(tbl_hbm, idx_hbm, out_hbm, idx_v, rows_a, rows_b,
                    sem_a, sem_b):
    wid = lax.axis_index("s") * SC_NC + lax.axis_index("c")
    ibase = wid * NG
    pltpu.sync_copy(idx_hbm.at[wid], idx_v)
    pltpu.make_async_copy(tbl_hbm.at[idx_v.at[0]], rows_a, sem_a).start()

    @pl.loop(0, (NG - 1) // 2)
    def _(i):
        g = 1 + 2 * i
        pltpu.make_async_copy(tbl_hbm.at[idx_v.at[g]], rows_b, sem_b).start()
        pltpu.make_async_copy(tbl_hbm.at[idx_v.at[g - 1]], rows_a, sem_a).wait()
        pltpu.sync_copy(rows_a, out_hbm.at[pl.ds((ibase + g - 1) * GCH, GCH)])
        pltpu.make_async_copy(tbl_hbm.at[idx_v.at[g + 1]], rows_a, sem_a).start()
        pltpu.make_async_copy(tbl_hbm.at[idx_v.at[g]], rows_b, sem_b).wait()
        pltpu.sync_copy(rows_b, out_hbm.at[pl.ds((ibase + g) * GCH, GCH)])

    pltpu.make_async_copy(tbl_hbm.at[idx_v.at[NG - 1]], rows_a, sem_a).wait()
    pltpu.sync_copy(rows_a, out_hbm.at[pl.ds((ibase + NG - 1) * GCH, GCH)])


def _sc_gather(table, gidx2d):
    k = pl.kernel(
        _sc_gather_body,
        out_type=jax.ShapeDtypeStruct((NROWTOT, CP), jnp.float32),
        mesh=plsc.VectorSubcoreMesh(core_axis_name="c", subcore_axis_name="s",
                                    num_cores=SC_NC, num_subcores=SC_NS),
        scratch_types=[
            pltpu.VMEM((NG, GCH), jnp.int32),
            pltpu.VMEM((GCH, CP), jnp.float32),
            pltpu.VMEM((GCH, CP), jnp.float32),
            pltpu.SemaphoreType.DMA,
            pltpu.SemaphoreType.DMA,
        ],
        compiler_params=pltpu.CompilerParams(use_tc_tiling_on_sc=True),
    )
    return k(table, gidx2d)


HCH = RCH // W


def _pad_body(src_ref, out_ref):
    v = src_ref[0, 0]
    vt = jnp.transpose(v, (0, 2, 1))
    out_ref[0, :, :C] = vt.reshape(RCH, C)
    out_ref[0, :, C:] = jnp.zeros((RCH, CP - C), jnp.float32)


def _pad128(gs_t):
    return pl.pallas_call(
        _pad_body,
        grid=(NIMG, NRB),
        in_specs=[pl.BlockSpec((1, 1, HCH, C, W),
                               lambda n, rb: (n // 3, n % 3, rb, 0, 0))],
        out_specs=pl.BlockSpec((1, RCH, CP), lambda n, rb: (n, rb, 0)),
        out_shape=jax.ShapeDtypeStruct((NIMG, HW, CP), jnp.float32),
    )(gs_t)


def _combine_body(src_ref, gat_ref, msk_ref, scal_ref, out_ref):
    n = pl.program_id(0)
    s = scal_ref[n, 0]
    r = scal_ref[n, 1]
    x = src_ref[0, :, :C]
    g = gat_ref[0, :, :C].astype(jnp.float32)
    m = msk_ref[0].astype(jnp.float32)
    out_ref[0] = ((x + (s * m) * g) * r).astype(jnp.bfloat16)


def _combine(src6, gath, mask, scal):
    def gmap(n, rb):
        i = n // 3
        j = n % 3
        p = i * 2 + jnp.where(j < 2, j, 0)
        return (p, rb, 0)

    return pl.pallas_call(
        _combine_body,
        grid=(NIMG, NRB),
        in_specs=[
            pl.BlockSpec((1, RCH, CP), lambda n, rb: (n, rb, 0)),
            pl.BlockSpec((1, RCH, CP), gmap),
            pl.BlockSpec((1, RCH, 1), gmap),
            pl.BlockSpec(memory_space=pltpu.SMEM),
        ],
        out_specs=pl.BlockSpec((1, RCH, C), lambda n, rb: (n, rb, 0)),
        out_shape=jax.ShapeDtypeStruct((NIMG, HW, C), jnp.bfloat16),
    )(src6, gath, mask, scal)


def _erf(x):
    a1, a2, a3, a4, a5 = (0.254829592, -0.284496736, 1.421413741,
                          -1.453152027, 1.061405429)
    sgn = jnp.sign(x)
    ax = jnp.abs(x)
    t = 1.0 / (1.0 + 0.3275911 * ax)
    poly = t * (a1 + t * (a2 + t * (a3 + t * (a4 + t * a5))))
    return sgn * (1.0 - poly * jnp.exp(-ax * ax))


_TAPS = tuple((dy, dx) for dy in (-1, 0, 1) for dx in (-1, 0, 1))


HWP = HW + 2 * PAD


def _build_shifted(x_hbm, comb_ref, sem, n):
    zpad = jnp.zeros((PAD, C), jnp.bfloat16)
    for s in range(3):
        comb_ref[s, pl.ds(0, PAD), :] = zpad
        comb_ref[s, pl.ds(PAD + HW, PAD), :] = zpad
    cp = pltpu.make_async_copy(
        x_hbm.at[n], comb_ref.at[1, pl.ds(PAD, HW)], sem)
    cp.start()
    cp.wait()
    wc = lax.broadcasted_iota(jnp.int32, (RCH, 1), 0) % W
    for t in range(NRB):
        q0 = PAD + t * RCH
        ext = comb_ref[1, pl.ds(q0 - 16, RCH + 32), :]
        sm = lax.slice_in_dim(ext, 15, 15 + RCH, axis=0)
        comb_ref[0, pl.ds(q0, RCH), :] = jnp.where(
            wc == 0, jnp.bfloat16(0), sm)
        sp = lax.slice_in_dim(ext, 17, 17 + RCH, axis=0)
        comb_ref[2, pl.ds(q0, RCH), :] = jnp.where(
            wc == W - 1, jnp.bfloat16(0), sp)


def _conv_accum(comb_ref, w_ref, bias_ref, rb):
    base = PAD + rb * RCH
    acc = jnp.broadcast_to(bias_ref[...].astype(jnp.float32), (RCH, C))
    for dy in (-1, 0, 1):
        for dx in (-1, 0, 1):
            t = (dy + 1) * 3 + (dx + 1)
            xs = comb_ref[dx + 1, pl.ds(base + dy * W, RCH), :]
            acc = acc + jax.lax.dot_general(
                xs, w_ref[t], (((1,), (0,)), ((), ())),
                preferred_element_type=jnp.float32)
    return acc


def _conv1_body(x_ref, w_ref, bias_ref, out_ref, comb_ref, sem):
    n = pl.program_id(0)
    rb = pl.program_id(1)

    @pl.when(rb == 0)
    def _():
        _build_shifted(x_ref, comb_ref, sem, n)

    acc = _conv_accum(comb_ref, w_ref, bias_ref, rb)
    gelu = acc * 0.5 * (1.0 + _erf(acc * 0.7071067811865476))
    out_ref[0] = gelu.astype(jnp.bfloat16)


def _conv2_body(x_ref, w_ref, bias_ref, out_ref, comb_ref, sem):
    n = pl.program_id(0)
    rb = pl.program_id(1)

    @pl.when(rb == 0)
    def _():
        _build_shifted(x_ref, comb_ref, sem, n)

    acc = _conv_accum(comb_ref, w_ref, bias_ref, rb)
    out_ref[0] = jnp.transpose(acc.reshape(HCH, W, C), (0, 2, 1))


def _conv(x, wt, bias, body, out_dtype, out_hcw):
    if out_hcw:
        out_specs = pl.BlockSpec((1, HCH, C, W), lambda n, rb: (n, rb, 0, 0))
        out_shape = jax.ShapeDtypeStruct((NIMG, H, C, W), out_dtype)
    else:
        out_specs = pl.BlockSpec((1, RCH, C), lambda n, rb: (n, rb, 0))
        out_shape = jax.ShapeDtypeStruct((NIMG, HW, C), out_dtype)
    return pl.pallas_call(
        body,
        grid=(NIMG, NRB),
        in_specs=[
            pl.BlockSpec(memory_space=pltpu.HBM),
            pl.BlockSpec((9, C, C), lambda n, rb: (0, 0, 0)),
            pl.BlockSpec((1, C), lambda n, rb: (0, 0)),
        ],
        out_specs=out_specs,
        out_shape=out_shape,
        scratch_shapes=[pltpu.VMEM((3, HWP, C), jnp.bfloat16),
                        pltpu.SemaphoreType.DMA],
    )(x, wt, bias)


def kernel(means, depths, gs_feats, intrinsics, extrinsics,
           conv1_w, conv1_b, conv2_w, conv2_b):
    del depths
    b, v, h, w, c = gs_feats.shape

    w2c = jnp.linalg.inv(extrinsics)
    pi = jnp.array([0, 0, 1, 1])
    pk = jnp.array([1, 2, 1, 2])
    w2c_p = w2c[pi, pk]
    intr_p = intrinsics[pi, pk]
    w2c_b = w2c_p.astype(jnp.bfloat16).astype(jnp.float32)
    intr_b = intr_p.astype(jnp.bfloat16).astype(jnp.float32)
    coef = jnp.zeros((NPAIR, 6, 4), jnp.float32)
    coef = coef.at[:, 0:3, :].set(w2c_b[:, 0:3, :])
    coef = coef.at[:, 3, 0:3].set(intr_b[:, 0, :])
    coef = coef.at[:, 4, 0:3].set(intr_b[:, 1, :])
    tgt_base = ((pi * v + pk) * HW).astype(jnp.float32)
    coef = coef.at[:, 5, 0].set(tgt_base)

    pts = means[:, :2].reshape(NPAIR, HW, 3).transpose(0, 2, 1)
    pts = pts.reshape(NPAIR, 3, ROWS, 128)

    gidx, maskf, cnt = _project(pts, coef)

    gs_t = jnp.transpose(gs_feats, (0, 1, 2, 4, 3))
    src_pad = _pad128(gs_t)
    table = src_pad.reshape(b * v * HW, CP)
    gidx2d = gidx.reshape(SC_NW, NG, GCH)
    gath = _sc_gather(table, gidx2d)

    frac = cnt[:, 0, 0] / float(HW)
    s_pair = GAMMA * frac
    s_img = jnp.stack([s_pair[0], s_pair[1], jnp.zeros(()),
                       s_pair[2], s_pair[3], jnp.zeros(())])
    r_img = 1.0 / (1.0 + s_img)
    scal = jnp.stack([s_img, r_img], axis=1)

    gath3 = gath.reshape(NPAIR, HW, CP)
    mask3 = maskf.reshape(NPAIR, HW, 1)
    comb = _combine(src_pad, gath3, mask3, scal)

    w1 = conv1_w.transpose(2, 3, 1, 0).reshape(9, C, C).astype(jnp.bfloat16)
    w2 = conv2_w.transpose(2, 3, 1, 0).reshape(9, C, C).astype(jnp.bfloat16)
    b1 = conv1_b.reshape(1, C)
    b2 = conv2_b.reshape(1, C)
    mid = _conv(comb, w1, b1, _conv1_body, jnp.bfloat16, out_hcw=False)
    out = _conv(mid, w2, b2, _conv2_body, jnp.float32, out_hcw=True)

    out5 = out.reshape(b, v, h, c, w)
    return jnp.transpose(out5, (0, 1, 2, 4, 3))


# speedup vs baseline: 1.3883x; 1.0031x over previous
import functools

import jax
import jax.numpy as jnp
from jax import lax
from jax.experimental import pallas as pl
from jax.experimental.pallas import tpu as pltpu
from jax.experimental.pallas import tpu_sc as plsc

H = 224
W = 224
HW = H * W
C = 96
GAMMA = 0.1
NPAIR = 4
ROWS = HW // 128
PAD = 256
RCH = 3136
NRB = HW // RCH
NIMG = 6
NCB = 4
CCH = HW // NCB

SC_NC = 2
SC_NS = 16
SC_NW = SC_NC * SC_NS
NROWTOT = NPAIR * HW
BPW = NROWTOT // SC_NW
GCH = 128
NG = BPW // GCH


def _bf(v):
    return v.astype(jnp.bfloat16).astype(jnp.float32)


def _proj_body(pts_ref, coef_ref, idx_ref, msk_ref, cnt_ref):
    p = pl.program_id(0)
    x = _bf(pts_ref[0, 0])
    y = _bf(pts_ref[0, 1])
    z = _bf(pts_ref[0, 2])

    def crow(r):
        return (coef_ref[p, r, 0], coef_ref[p, r, 1], coef_ref[p, r, 2],
                coef_ref[p, r, 3])

    a0, a1, a2, a3 = crow(0)
    camx = a0 * x + a1 * y + a2 * z + a3
    a0, a1, a2, a3 = crow(1)
    camy = a0 * x + a1 * y + a2 * z + a3
    a0, a1, a2, a3 = crow(2)
    camz = a0 * x + a1 * y + a2 * z + a3
    denom = camz + 1e-8
    px_ = _bf(camx / denom)
    py_ = _bf(camy / denom)
    pz_ = _bf(camz / denom)
    i00, i01, i02, _ = crow(3)
    ndcx = i00 * px_ + i01 * py_ + i02 * pz_
    i10, i11, i12, _ = crow(4)
    ndcy = i10 * px_ + i11 * py_ + i12 * pz_
    valid = ((ndcx >= 0.0) & (ndcx < 1.0) & (ndcy >= 0.0) & (ndcy < 1.0)
             & (camz >= 0.0))
    pxi = jnp.clip(jnp.floor(ndcx * W).astype(jnp.int32), 0, W - 1)
    pyi = jnp.clip(jnp.floor(ndcy * H).astype(jnp.int32), 0, H - 1)
    base = coef_ref[p, 5, 0].astype(jnp.int32)
    idx_ref[...] = (base + pyi * W + pxi)[None]
    mf = valid.astype(jnp.float32)
    msk_ref[...] = mf[None]
    cnt_ref[...] = jnp.broadcast_to(jnp.sum(mf), (1, 1, 128))


def _project(pts, coef):
    return pl.pallas_call(
        _proj_body,
        grid=(NPAIR,),
        in_specs=[
            pl.BlockSpec((1, 3, ROWS, 128), lambda p: (p, 0, 0, 0)),
            pl.BlockSpec(memory_space=pltpu.SMEM),
        ],
        out_specs=[
            pl.BlockSpec((1, ROWS, 128), lambda p: (p, 0, 0)),
            pl.BlockSpec((1, ROWS, 128), lambda p: (p, 0, 0)),
            pl.BlockSpec((1, 1, 128), lambda p: (p, 0, 0)),
        ],
        out_shape=[
            jax.ShapeDtypeStruct((NPAIR, ROWS, 128), jnp.int32),
            jax.ShapeDtypeStruct((NPAIR, ROWS, 128), jnp.float32),
            jax.ShapeDtypeStruct((NPAIR, 1, 128), jnp.float32),
        ],
    )(pts, coef)


CP = 128


def _sc_gather_body(tbl_hbm, idx_hbm, out_hbm, idx_v, rows_a, rows_b,
                    sem_a, sem_b):
    wid = lax.axis_index("s") * SC_NC + lax.axis_index("c")
    ibase = wid * NG
    pltpu.sync_copy(idx_hbm.at[wid], idx_v)
    pltpu.make_async_copy(tbl_hbm.at[idx_v.at[0]], rows_a, sem_a).start()

    @pl.loop(0, (NG - 1) // 2)
    def _(i):
        g = 1 + 2 * i
        pltpu.make_async_copy(tbl_hbm.at[idx_v.at[g]], rows_b, sem_b).start()
        pltpu.make_async_copy(tbl_hbm.at[idx_v.at[g - 1]], rows_a, sem_a).wait()
        pltpu.sync_copy(rows_a, out_hbm.at[pl.ds((ibase + g - 1) * GCH, GCH)])
        pltpu.make_async_copy(tbl_hbm.at[idx_v.at[g + 1]], rows_a, sem_a).start()
        pltpu.make_async_copy(tbl_hbm.at[idx_v.at[g]], rows_b, sem_b).wait()
        pltpu.sync_copy(rows_b, out_hbm.at[pl.ds((ibase + g) * GCH, GCH)])

    pltpu.make_async_copy(tbl_hbm.at[idx_v.at[NG - 1]], rows_a, sem_a).wait()
    pltpu.sync_copy(rows_a, out_hbm.at[pl.ds((ibase + NG - 1) * GCH, GCH)])


def _sc_gather(table, gidx2d):
    k = pl.kernel(
        _sc_gather_body,
        out_type=jax.ShapeDtypeStruct((NROWTOT, CP), jnp.float32),
        mesh=plsc.VectorSubcoreMesh(core_axis_name="c", subcore_axis_name="s",
                                    num_cores=SC_NC, num_subcores=SC_NS),
        scratch_types=[
            pltpu.VMEM((NG, GCH), jnp.int32),
            pltpu.VMEM((GCH, CP), jnp.float32),
            pltpu.VMEM((GCH, CP), jnp.float32),
            pltpu.SemaphoreType.DMA,
            pltpu.SemaphoreType.DMA,
        ],
        compiler_params=pltpu.CompilerParams(use_tc_tiling_on_sc=True),
    )
    return k(table, gidx2d)


HCH = RCH // W


def _pad_body(src_ref, out_ref):
    v = src_ref[0, 0]
    vt = jnp.transpose(v, (0, 2, 1))
    out_ref[0, :, :C] = vt.reshape(RCH, C)
    out_ref[0, :, C:] = jnp.zeros((RCH, CP - C), jnp.float32)


def _pad128(gs_t):
    return pl.pallas_call(
        _pad_body,
        grid=(NIMG, NRB),
        in_specs=[pl.BlockSpec((1, 1, HCH, C, W),
                               lambda n, rb: (n // 3, n % 3, rb, 0, 0))],
        out_specs=pl.BlockSpec((1, RCH, CP), lambda n, rb: (n, rb, 0)),
        out_shape=jax.ShapeDtypeStruct((NIMG, HW, CP), jnp.float32),
    )(gs_t)


def _combine_body(src_ref, gat_ref, msk_ref, scal_ref, out_ref):
    n = pl.program_id(0)
    s = scal_ref[n, 0]
    r = scal_ref[n, 1]
    x = src_ref[0, :, :C]
    g = gat_ref[0, :, :C].astype(jnp.float32)
    m = msk_ref[0].astype(jnp.float32)
    out_ref[0] = ((x + (s * m) * g) * r).astype(jnp.bfloat16)


def _combine(src6, gath, mask, scal):
    def gmap(n, rb):
        i = n // 3
        j = n % 3
        p = i * 2 + jnp.where(j < 2, j, 0)
        return (p, rb, 0)

    return pl.pallas_call(
        _combine_body,
        grid=(NIMG, NCB),
        in_specs=[
            pl.BlockSpec((1, CCH, CP), lambda n, rb: (n, rb, 0)),
            pl.BlockSpec((1, CCH, CP), gmap),
            pl.BlockSpec((1, CCH, 1), gmap),
            pl.BlockSpec(memory_space=pltpu.SMEM),
        ],
        out_specs=pl.BlockSpec((1, CCH, C), lambda n, rb: (n, rb, 0)),
        out_shape=jax.ShapeDtypeStruct((NIMG, HW, C), jnp.bfloat16),
    )(src6, gath, mask, scal)


def _erf(x):
    a1, a2, a3, a4, a5 = (0.254829592, -0.284496736, 1.421413741,
                          -1.453152027, 1.061405429)
    sgn = jnp.sign(x)
    ax = jnp.abs(x)
    t = 1.0 / (1.0 + 0.3275911 * ax)
    poly = t * (a1 + t * (a2 + t * (a3 + t * (a4 + t * a5))))
    return sgn * (1.0 - poly * jnp.exp(-ax * ax))


_TAPS = tuple((dy, dx) for dy in (-1, 0, 1) for dx in (-1, 0, 1))


HWP = HW + 2 * PAD


def _build_shifted(x_hbm, comb_ref, sem, n):
    zpad = jnp.zeros((PAD, C), jnp.bfloat16)
    for s in range(3):
        comb_ref[s, pl.ds(0, PAD), :] = zpad
        comb_ref[s, pl.ds(PAD + HW, PAD), :] = zpad
    cp = pltpu.make_async_copy(
        x_hbm.at[n], comb_ref.at[1, pl.ds(PAD, HW)], sem)
    cp.start()
    cp.wait()
    wc = lax.broadcasted_iota(jnp.int32, (RCH, 1), 0) % W
    for t in range(NRB):
        q0 = PAD + t * RCH
        ext = comb_ref[1, pl.ds(q0 - 16, RCH + 32), :]
        sm = lax.slice_in_dim(ext, 15, 15 + RCH, axis=0)
        comb_ref[0, pl.ds(q0, RCH), :] = jnp.where(
            wc == 0, jnp.bfloat16(0), sm)
        sp = lax.slice_in_dim(ext, 17, 17 + RCH, axis=0)
        comb_ref[2, pl.ds(q0, RCH), :] = jnp.where(
            wc == W - 1, jnp.bfloat16(0), sp)


def _conv_accum(comb_ref, w_ref, bias_ref, rb):
    base = PAD + rb * RCH
    acc = jnp.broadcast_to(bias_ref[...].astype(jnp.float32), (RCH, C))
    for dy in (-1, 0, 1):
        for dx in (-1, 0, 1):
            t = (dy + 1) * 3 + (dx + 1)
            xs = comb_ref[dx + 1, pl.ds(base + dy * W, RCH), :]
            acc = acc + jax.lax.dot_general(
                xs, w_ref[t], (((1,), (0,)), ((), ())),
                preferred_element_type=jnp.float32)
    return acc


def _conv1_body(x_ref, w_ref, bias_ref, out_ref, comb_ref, sem):
    n = pl.program_id(0)
    rb = pl.program_id(1)

    @pl.when(rb == 0)
    def _():
        _build_shifted(x_ref, comb_ref, sem, n)

    acc = _conv_accum(comb_ref, w_ref, bias_ref, rb)
    gelu = acc * 0.5 * (1.0 + _erf(acc * 0.7071067811865476))
    out_ref[0] = gelu.astype(jnp.bfloat16)


def _conv2_body(x_ref, w_ref, bias_ref, out_ref, comb_ref, sem):
    n = pl.program_id(0)
    rb = pl.program_id(1)

    @pl.when(rb == 0)
    def _():
        _build_shifted(x_ref, comb_ref, sem, n)

    acc = _conv_accum(comb_ref, w_ref, bias_ref, rb)
    out_ref[0] = jnp.transpose(acc.reshape(HCH, W, C), (0, 2, 1))


def _conv(x, wt, bias, body, out_dtype, out_hcw):
    if out_hcw:
        out_specs = pl.BlockSpec((1, HCH, C, W), lambda n, rb: (n, rb, 0, 0))
        out_shape = jax.ShapeDtypeStruct((NIMG, H, C, W), out_dtype)
    else:
        out_specs = pl.BlockSpec((1, RCH, C), lambda n, rb: (n, rb, 0))
        out_shape = jax.ShapeDtypeStruct((NIMG, HW, C), out_dtype)
    return pl.pallas_call(
        body,
        grid=(NIMG, NRB),
        in_specs=[
            pl.BlockSpec(memory_space=pltpu.HBM),
            pl.BlockSpec((9, C, C), lambda n, rb: (0, 0, 0)),
            pl.BlockSpec((1, C), lambda n, rb: (0, 0)),
        ],
        out_specs=out_specs,
        out_shape=out_shape,
        scratch_shapes=[pltpu.VMEM((3, HWP, C), jnp.bfloat16),
                        pltpu.SemaphoreType.DMA],
    )(x, wt, bias)


def kernel(means, depths, gs_feats, intrinsics, extrinsics,
           conv1_w, conv1_b, conv2_w, conv2_b):
    del depths
    b, v, h, w, c = gs_feats.shape

    w2c = jnp.linalg.inv(extrinsics)
    pi = jnp.array([0, 0, 1, 1])
    pk = jnp.array([1, 2, 1, 2])
    w2c_p = w2c[pi, pk]
    intr_p = intrinsics[pi, pk]
    w2c_b = w2c_p.astype(jnp.bfloat16).astype(jnp.float32)
    intr_b = intr_p.astype(jnp.bfloat16).astype(jnp.float32)
    coef = jnp.zeros((NPAIR, 6, 4), jnp.float32)
    coef = coef.at[:, 0:3, :].set(w2c_b[:, 0:3, :])
    coef = coef.at[:, 3, 0:3].set(intr_b[:, 0, :])
    coef = coef.at[:, 4, 0:3].set(intr_b[:, 1, :])
    tgt_base = ((pi * v + pk) * HW).astype(jnp.float32)
    coef = coef.at[:, 5, 0].set(tgt_base)

    pts = means[:, :2].reshape(NPAIR, HW, 3).transpose(0, 2, 1)
    pts = pts.reshape(NPAIR, 3, ROWS, 128)

    gidx, maskf, cnt = _project(pts, coef)

    gs_t = jnp.transpose(gs_feats, (0, 1, 2, 4, 3))
    src_pad = _pad128(gs_t)
    table = src_pad.reshape(b * v * HW, CP)
    gidx2d = gidx.reshape(SC_NW, NG, GCH)
    gath = _sc_gather(table, gidx2d)

    frac = cnt[:, 0, 0] / float(HW)
    s_pair = GAMMA * frac
    s_img = jnp.stack([s_pair[0], s_pair[1], jnp.zeros(()),
                       s_pair[2], s_pair[3], jnp.zeros(())])
    r_img = 1.0 / (1.0 + s_img)
    scal = jnp.stack([s_img, r_img], axis=1)

    gath3 = gath.reshape(NPAIR, HW, CP)
    mask3 = maskf.reshape(NPAIR, HW, 1)
    comb = _combine(src_pad, gath3, mask3, scal)

    w1 = conv1_w.transpose(2, 3, 1, 0).reshape(9, C, C).astype(jnp.bfloat16)
    w2 = conv2_w.transpose(2, 3, 1, 0).reshape(9, C, C).astype(jnp.bfloat16)
    b1 = conv1_b.reshape(1, C)
    b2 = conv2_b.reshape(1, C)
    mid = _conv(comb, w1, b1, _conv1_body, jnp.bfloat16, out_hcw=False)
    out = _conv(mid, w2, b2, _conv2_body, jnp.float32, out_hcw=True)

    out5 = out.reshape(b, v, h, c, w)
    return jnp.transpose(out5, (0, 1, 2, 4, 3))
